# baseline (speedup 1.0000x reference)
import jax
import jax.numpy as jnp
from jax import lax
from jax.experimental import pallas as pl
from jax.experimental.pallas import tpu as pltpu, tpu_sc as plsc

N_CLASSES = 21
THRESHOLD = 0.5
NEGPOS_RATIO = 3
VAR0 = 0.1
VAR1 = 0.2

B = 8
P = 8732
O = 16
NC = 2
NS = 16
L = 16
NB = B // NC
VPS = 560
OWN = 552
DMAR = 560
TSTART = 8724
NV = VPS // L
C = N_CLASSES

_F = jnp.float32
_I = jnp.int32


def _iota():
  return lax.iota(_I, L)


def _spl_f(x):
  return jnp.full((L,), x, _F)


def _spl_i(x):
  return jnp.full((L,), x, _I)


def _vln(x):
  x = jnp.maximum(x, _spl_f(1e-30))
  bits = plsc.bitcast(x, _I)
  e = lax.shift_right_logical(bits, _spl_i(23)) - _spl_i(127)
  m = plsc.bitcast((bits & _spl_i(0x007FFFFF)) | _spl_i(127 << 23), _F)
  big = m > _spl_f(1.4142135623730951)
  m = jnp.where(big, m * _spl_f(0.5), m)
  e = e + jnp.where(big, _spl_i(1), _spl_i(0))
  z = (m - _spl_f(1.0)) / (m + _spl_f(1.0))
  z2 = z * z
  p = _spl_f(1.0) + z2 * (_spl_f(1.0 / 3) + z2 * (_spl_f(1.0 / 5)
                                                  + z2 * _spl_f(1.0 / 7)))
  return _spl_f(2.0) * z * p + e.astype(_F) * _spl_f(0.6931471805599453)


def _lane_extract_i(vec, lane_splat):
  return jnp.max(jnp.where(_iota() == lane_splat, vec, _spl_i(-2**31 + 1)))


LOFF = 0
POFF = B * 8960 * 4
TOFF = POFF + 8960 * 4


def _sc_body(aux_hbm, conf_hbm, out_hbm,
             conf_v, loc_v, pri_v, tgt_v, trd_v, bto_v, bti_v,
             amv_v, ami_v, msk_v, hist_v, red_v, totf_v, sta_v, fin_v,
             bpi_v, thr_v, scr_v, scri_v,
             amv_s, ami_s, bpi_s, sta_s, his_s, thr_s, fin_s):
  cid = lax.axis_index("c")
  sid = lax.axis_index("s")
  p0 = sid * OWN
  start = pl.multiple_of(jnp.minimum(p0, 8176), 8)
  iot = _iota()
  p0v = jnp.full((L,), p0, _I)
  dlv = jnp.full((L,), p0 - start, _I)
  cidv = jnp.full((L,), cid, _I)

  def prow(pvec, gidx):
    return jnp.minimum(pvec + dlv, _spl_i(555))

  def spl(x, dtype=_I):
    return jnp.full((L,), x, dtype)

  def gat(ref, idx):
    return plsc.load_gather(ref, [idx])

  def sca(ref, idx, val, mask=None):
    plsc.store_scatter(ref, [idx], val, mask=mask)

  pltpu.sync_copy(aux_hbm.at[pl.ds(POFF + p0 * 4, VPS * 4)], pri_v)
  pltpu.sync_copy(aux_hbm.at[pl.ds(TOFF + cid * NB * O * 5, NB * O * 5)],
                  tgt_v)

  def truthd_body(b, _):
    tb = spl(b * O * 5) + iot * _spl_i(5)
    tx1 = gat(tgt_v, tb)
    ty1 = gat(tgt_v, tb + _spl_i(1))
    tx2 = gat(tgt_v, tb + _spl_i(2))
    ty2 = gat(tgt_v, tb + _spl_i(3))
    lab = gat(tgt_v, tb + _spl_i(4))
    dst = spl(b * 8 * L) + iot
    sca(trd_v, dst, tx1)
    sca(trd_v, dst + _spl_i(L), ty1)
    sca(trd_v, dst + _spl_i(2 * L), tx2)
    sca(trd_v, dst + _spl_i(3 * L), ty2)
    sca(trd_v, dst + _spl_i(4 * L), (tx2 - tx1) * (ty2 - ty1))
    sca(trd_v, dst + _spl_i(5 * L), lab)
    return 0
  lax.fori_loop(0, NB, truthd_body, 0)

  def jac_b(b, _):
    tbase = spl(b * 8 * L)
    abase = spl(b * O * L)

    def init_o(o, _):
      dst = abase + spl(o * L) + iot
      sca(amv_v, dst, _spl_f(-1.0))
      sca(ami_v, dst, _spl_f(0.0))
      return 0
    lax.fori_loop(0, O, init_o, 0)

    def jac_v(v, _):
      pvec = v * L + iot
      gidx = p0v + pvec
      gidxf = gidx.astype(_F)
      validv = (pvec < _spl_i(OWN)) & (gidx < _spl_i(P))
      row = prow(pvec, gidx)
      p4 = pvec * _spl_i(4)
      pcx = gat(pri_v, p4)
      pcy = gat(pri_v, p4 + _spl_i(1))
      pw = gat(pri_v, p4 + _spl_i(2))
      ph = gat(pri_v, p4 + _spl_i(3))
      pw2 = pw * _spl_f(0.5)
      ph2 = ph * _spl_f(0.5)
      px1 = pcx - pw2
      px2 = pcx + pw2
      py1 = pcy - ph2
      py2 = pcy + ph2
      areab = pw * ph
      bto = _spl_f(-1.0)
      bti = _spl_f(0.0)
      for o in range(O):
        tx1 = gat(trd_v, tbase + _spl_i(o))
        ty1 = gat(trd_v, tbase + _spl_i(L + o))
        tx2 = gat(trd_v, tbase + _spl_i(2 * L + o))
        ty2 = gat(trd_v, tbase + _spl_i(3 * L + o))
        ta = gat(trd_v, tbase + _spl_i(4 * L + o))
        iw = jnp.maximum(jnp.minimum(px2, tx2) - jnp.maximum(px1, tx1),
                         _spl_f(0.0))
        ih = jnp.maximum(jnp.minimum(py2, ty2) - jnp.maximum(py1, ty1),
                         _spl_f(0.0))
        inter = iw * ih
        ov = inter / (ta + areab - inter)
        m = ov > bto
        bto = jnp.where(m, ov, bto)
        bti = jnp.where(m, _spl_f(o), bti)
        adst = abase + spl(o * L) + iot
        av = gat(amv_v, adst)
        ai = gat(ami_v, adst)
        mu = (ov > av) & validv
        sca(amv_v, adst, jnp.where(mu, ov, av))
        sca(ami_v, adst, jnp.where(mu, gidxf, ai))
      bdst = spl(b * VPS) + pvec
      sca(bto_v, bdst, bto)
      sca(bti_v, bdst, bti)
      return 0
    lax.fori_loop(0, NV, jac_v, 0)
    return 0
  lax.fori_loop(0, NB, jac_b, 0)

  def pub_am(b, _):
    pltpu.sync_copy(amv_v.at[pl.ds(b * O * L, O * L)],
                    amv_s.at[pl.ds(b * NS * O * L + sid * O * L, O * L)])
    pltpu.sync_copy(ami_v.at[pl.ds(b * O * L, O * L)],
                    ami_s.at[pl.ds(b * NS * O * L + sid * O * L, O * L)])
    return 0
  lax.fori_loop(0, NB, pub_am, 0)
  plsc.subcore_barrier()

  @pl.when(sid < NB)
  def _():
    jb = sid
    pltpu.sync_copy(amv_s.at[pl.ds(jb * NS * O * L, NS * O * L)], amv_v)
    pltpu.sync_copy(ami_s.at[pl.ds(jb * NS * O * L, NS * O * L)], ami_v)
    bvec = _spl_f(0.0)
    for o in range(O):
      def red_s(s2, carry):
        rv, ri = carry
        src = spl(s2 * O * L + o * L) + iot
        av = gat(amv_v, src)
        ai = gat(ami_v, src)
        better = (av > rv) | ((av == rv) & (ai < ri))
        return jnp.where(better, av, rv), jnp.where(better, ai, ri)
      rv, ri = lax.fori_loop(0, NS, red_s, (_spl_f(-1.0), _spl_f(0.0)))
      mx = jnp.max(rv)
      cand = jnp.where(rv == jnp.full((L,), mx, _F), ri, _spl_f(3e38))
      bpio = jnp.min(cand)
      bvec = jnp.where(iot == _spl_i(o), jnp.full((L,), bpio, _F), bvec)
    sca(scr_v, iot, bvec)
    pltpu.sync_copy(scr_v.at[pl.ds(0, L)], bpi_s.at[pl.ds(jb * L, L)])
  plsc.subcore_barrier()

  pltpu.sync_copy(bpi_s, bpi_v)

  def ph2_b(b, carry):
    accl_t, accce_t = carry
    gb = cid * NB + b
    pltpu.sync_copy(conf_hbm.at[gb, pl.ds(start, DMAR), :], conf_v)
    pltpu.sync_copy(aux_hbm.at[pl.ds(gb * 35840 + p0 * 4, VPS * 4)], loc_v)
    tbase = spl(b * 8 * L)
    bbase = spl(b * VPS)
    bv = spl(b)
    bpirow = gat(bpi_v, spl(b * L) + iot).astype(_I)
    localp = bpirow - p0v
    inr = (localp >= _spl_i(0)) & (localp < _spl_i(OWN))
    localc = jnp.minimum(jnp.maximum(localp, _spl_i(0)),
                         _spl_i(VPS - 1)) + bbase
    for o in range(O):
      mo = inr & (iot == _spl_i(o))
      sca(bto_v, localc, _spl_f(2.0), mask=mo)
      sca(bti_v, localc, _spl_f(o), mask=mo)

    def ph2_v(v, carry):
      accl, accce, accnp = carry
      pvec = v * L + iot
      gidx = p0v + pvec
      bto = gat(bto_v, bbase + pvec)
      bti = gat(bti_v, bbase + pvec)
      pos = bto >= _spl_f(THRESHOLD)
      valid = (pvec < _spl_i(OWN)) & (gidx < _spl_i(P))
      row = prow(pvec, gidx)
      btii = bti.astype(_I)
      mx1 = gat(trd_v, tbase + btii)
      my1 = gat(trd_v, tbase + _spl_i(L) + btii)
      mx2 = gat(trd_v, tbase + _spl_i(2 * L) + btii)
      my2 = gat(trd_v, tbase + _spl_i(3 * L) + btii)
      p4 = pvec * _spl_i(4)
      pcx = gat(pri_v, p4)
      pcy = gat(pri_v, p4 + _spl_i(1))
      pw = gat(pri_v, p4 + _spl_i(2))
      ph = gat(pri_v, p4 + _spl_i(3))
      gx = ((mx1 + mx2) * _spl_f(0.5) - pcx) / (pw * _spl_f(VAR0))
      gy = ((my1 + my2) * _spl_f(0.5) - pcy) / (ph * _spl_f(VAR0))
      gw = _vln((mx2 - mx1) / pw) * _spl_f(1.0 / VAR1)
      gh = _vln((my2 - my1) / ph) * _spl_f(1.0 / VAR1)
      sl = _spl_f(0.0)
      for c4, g in ((0, gx), (1, gy), (2, gw), (3, gh)):
        lc = gat(loc_v, p4 + _spl_i(c4))
        d = lc - g
        ad = jnp.abs(d)
        sl = sl + jnp.where(ad < _spl_f(1.0), _spl_f(0.5) * d * d,
                            ad - _spl_f(0.5))
      cls_vals = [plsc.load_gather(conf_v, [row, _spl_i(c)])
                  for c in range(C)]
      mrow = cls_vals[0]
      for cv in cls_vals[1:]:
        mrow = jnp.maximum(mrow, cv)
      ssum = _spl_f(0.0)
      for cv in cls_vals:
        ssum = ssum + jnp.exp(cv - mrow)
      lse = _vln(ssum) + mrow
      cls = jnp.where(pos, _spl_i(1), _spl_i(0))
      quirk = (gidx == _spl_i(0)) & (cidv == _spl_i(0)) & \
          (bv == _spl_i(0))
      cls = jnp.where(quirk, _spl_i(1), cls)
      g0 = plsc.load_gather(conf_v, [row, cls])
      ce = lse - g0
      posv = pos & valid
      accl = accl + jnp.where(posv, sl, _spl_f(0.0))
      accce = accce + jnp.where(posv, ce, _spl_f(0.0))
      accnp = accnp + jnp.where(posv, _spl_f(1.0), _spl_f(0.0))
      sca(msk_v, bbase + pvec,
          jnp.where(valid & jnp.logical_not(pos),
                    jnp.maximum(ce, _spl_f(0.0)), _spl_f(0.0)))
      return accl, accce, accnp

    accl, accce, accnp = lax.fori_loop(
        0, NV, ph2_v, (_spl_f(0.0), _spl_f(0.0), _spl_f(0.0)))
    sca(scr_v, spl(b * L) + iot, accnp)
    return accl_t + accl, accce_t + accce

  accl_t, accce_t = lax.fori_loop(0, NB, ph2_b, (_spl_f(0.0), _spl_f(0.0)))
  sca(scr_v, spl(NB * L) + iot, accl_t)
  sca(scr_v, spl((NB + 1) * L) + iot, accce_t)
  pltpu.sync_copy(scr_v.at[pl.ds(0, 6 * L)], sta_s.at[pl.ds(sid * 6 * L,
                                                            6 * L)])

  ones = _spl_i(1)

  def round_r(r, _):
    shift = _spl_i(24) - r * _spl_i(8)
    shift_hi = jnp.minimum(shift + _spl_i(8), _spl_i(31))
    is0v = jnp.full((L,), r == 0, jnp.bool_)

    def hist_b(b, _):
      hbase = spl(b * 256)
      def zero_h(i, _):
        sca(hist_v, hbase + spl(i * L) + iot, _spl_i(0))
        return 0
      lax.fori_loop(0, 256 // L, zero_h, 0)
      pref = gat(thr_v, spl(b * L))
      bbase = spl(b * VPS)
      def hist_vv(v, _):
        mvec = gat(msk_v, bbase + v * L + iot)
        bits = plsc.bitcast(mvec, _I)
        match = is0v | (lax.shift_right_logical(bits, shift_hi) == pref)
        bucket = lax.shift_right_logical(bits, shift) & _spl_i(255)
        plsc.addupdate_scatter(hist_v, [hbase + bucket], ones, mask=match)
        return 0
      lax.fori_loop(0, NV, hist_vv, 0)
      pltpu.sync_copy(hist_v.at[pl.ds(b * 256, 256)],
                      his_s.at[pl.ds(b * NS * 256 + sid * 256, 256)])
      return 0
    lax.fori_loop(0, NB, hist_b, 0)
    plsc.subcore_barrier()

    @pl.when(sid < NB)
    def _():
      jb = sid
      jbv = jnp.full((L,), jb, _I)
      pltpu.sync_copy(his_s.at[pl.ds(jb * NS * 256, NS * 256)], red_v)
      pltpu.sync_copy(sta_s, sta_v)
      def nps(s2, acc):
        return acc + gat(sta_v, spl(s2 * 6 * L + jb * L) + iot)
      npsum = lax.fori_loop(0, NS, nps, _spl_f(0.0))
      npb = jnp.sum(npsum)
      k = jnp.minimum(npb * jnp.float32(NEGPOS_RATIO), jnp.float32(P - 1))
      ki = k.astype(_I)
      pref_old = gat(thr_v, jbv * _spl_i(L))
      kk_old = gat(thr_v, jbv * _spl_i(L) + _spl_i(1))
      kkv = jnp.where(is0v, jnp.full((L,), ki, _I), kk_old)
      def tot_i(i, _):
        def tadd(s2, acc):
          return acc + gat(red_v, spl(s2 * 256 + i * L) + iot)
        t = lax.fori_loop(0, NS, tadd, _spl_i(0))
        sca(totf_v, spl(i * L) + iot, t)
        return 0
      lax.fori_loop(0, 256 // L, tot_i, 0)
      def chunk(ic, carry):
        crun, found, gsel, kksel = carry
        i = _spl_i(255 // L) - ic
        tvec = gat(totf_v, i * _spl_i(L) + iot)
        rev = lax.rev(tvec, (0,))
        cs = plsc.cumsum(rev)
        tot = jnp.full((L,), jnp.max(cs), _I)
        candv = crun + cs
        hitm = candv >= kkv
        anyhit = jnp.full((L,), jnp.max(hitm.astype(_I)), _I) > _spl_i(0)
        ffs = plsc.all_reduce_ffs(hitm)
        gcand = i * _spl_i(L) + _spl_i(L - 1) - ffs
        cnt_at = jnp.full((L,), _lane_extract_i(rev, ffs), _I)
        cum_at = jnp.full((L,), _lane_extract_i(candv, ffs), _I)
        kkcand = kkv - (cum_at - cnt_at)
        take = jnp.logical_not(found) & anyhit
        gsel = jnp.where(take, gcand, gsel)
        kksel = jnp.where(take, kkcand, kksel)
        found = found | anyhit
        crun = crun + tot
        return crun, found, gsel, kksel
      crun, found, gsel, kksel = lax.fori_loop(
          0, 256 // L, chunk,
          (_spl_i(0), jnp.full((L,), False, jnp.bool_), _spl_i(0),
           _spl_i(0)))
      pnew = jnp.where(is0v, gsel,
                       lax.shift_left(pref_old, _spl_i(8)) | gsel)
      outv = jnp.where(iot == _spl_i(0), pnew,
                       jnp.where(iot == _spl_i(1), kksel, _spl_i(0)))
      sca(scri_v, iot, outv)
      pltpu.sync_copy(scri_v.at[pl.ds(0, L)], thr_s.at[pl.ds(jb * L, L)])
    plsc.subcore_barrier()
    pltpu.sync_copy(thr_s, thr_v)
    return 0

  def zthr(b, _):
    sca(thr_v, spl(b * L) + iot, _spl_i(0))
    return 0
  lax.fori_loop(0, NB, zthr, 0)
  lax.fori_loop(0, 4, round_r, 0)

  def fin_b(b, _):
    tbits = gat(thr_v, spl(b * L))
    tval = plsc.bitcast(tbits, _F)
    bbase = spl(b * VPS)
    def fin_vv(v, acc):
      mvec = gat(msk_v, bbase + v * L + iot)
      return acc + jnp.where(mvec > tval, mvec, _spl_f(0.0))
    acc = lax.fori_loop(0, NV, fin_vv, _spl_f(0.0))
    sca(scr_v, spl(b * L) + iot, acc)
    return 0
  lax.fori_loop(0, NB, fin_b, 0)
  pltpu.sync_copy(scr_v.at[pl.ds(0, NB * L)],
                  fin_s.at[pl.ds(sid * NB * L, NB * L)])
  plsc.subcore_barrier()

  @pl.when(sid == 0)
  def _():
    pltpu.sync_copy(fin_s, fin_v)
    pltpu.sync_copy(sta_s, sta_v)
    def sacc(s2, carry):
      lsum, cesum = carry
      return (lsum + gat(sta_v, spl(s2 * 6 * L + NB * L) + iot),
              cesum + gat(sta_v, spl(s2 * 6 * L + (NB + 1) * L) + iot))
    lsum, cesum = lax.fori_loop(0, NS, sacc, (_spl_f(0.0), _spl_f(0.0)))
    loss_l = jnp.sum(lsum)
    loss_c = jnp.sum(cesum)
    nsum = jnp.float32(0.0)
    for jb in range(NB):
      def npacc(s2, acc):
        return acc + gat(sta_v, spl(s2 * 6 * L + jb * L) + iot)
      npv = lax.fori_loop(0, NS, npacc, _spl_f(0.0))
      nsum = nsum + jnp.sum(npv)
      def facc(s2, acc):
        return acc + gat(fin_v, spl(s2 * NB * L + jb * L) + iot)
      sv = lax.fori_loop(0, NS, facc, _spl_f(0.0))
      tbits = gat(thr_v, _spl_i(jb * L))
      kkrem = gat(thr_v, _spl_i(jb * L + 1))
      tval = plsc.bitcast(tbits, _F)
      loss_c = loss_c + jnp.sum(sv) + jnp.max(kkrem).astype(_F) \
          * jnp.max(tval)
    ovec = jnp.where(iot == _spl_i(0), jnp.full((L,), loss_l, _F),
                     jnp.where(iot == _spl_i(1), jnp.full((L,), loss_c, _F),
                               jnp.where(iot == _spl_i(2),
                                         jnp.full((L,), nsum, _F),
                                         _spl_f(0.0))))
    sca(scr_v, iot, ovec)
    pltpu.sync_copy(scr_v.at[pl.ds(0, L)], out_hbm.at[pl.ds(cid * L, L)])


@jax.jit
def _run(aux, conf_p):
  mesh = plsc.VectorSubcoreMesh(core_axis_name="c", subcore_axis_name="s")
  f = pl.kernel(
      _sc_body,
      out_type=jax.ShapeDtypeStruct((NC * L,), _F),
      mesh=mesh,
      compiler_params=pltpu.CompilerParams(needs_layout_passes=False),
      scratch_types=[
          pltpu.VMEM((DMAR, C), _F),
          pltpu.VMEM((VPS * 4,), _F),
          pltpu.VMEM((VPS * 4,), _F),
          pltpu.VMEM((NB * O * 5,), _F),
          pltpu.VMEM((NB * 8 * L,), _F),
          pltpu.VMEM((NB * VPS,), _F),
          pltpu.VMEM((NB * VPS,), _F),
          pltpu.VMEM((NS * O * L,), _F),
          pltpu.VMEM((NS * O * L,), _F),
          pltpu.VMEM((NB * VPS,), _F),
          pltpu.VMEM((NB * 256,), _I),
          pltpu.VMEM((NS * 256,), _I),
          pltpu.VMEM((256,), _I),
          pltpu.VMEM((NS * 6 * L,), _F),
          pltpu.VMEM((NS * NB * L,), _F),
          pltpu.VMEM((NB * L,), _F),
          pltpu.VMEM((NB * L,), _I),
          pltpu.VMEM((8 * L,), _F),
          pltpu.VMEM((2 * L,), _I),
          pltpu.VMEM_SHARED((NB * NS * O * L,), _F),
          pltpu.VMEM_SHARED((NB * NS * O * L,), _F),
          pltpu.VMEM_SHARED((NB * L,), _F),
          pltpu.VMEM_SHARED((NS * 6 * L,), _F),
          pltpu.VMEM_SHARED((NB * NS * 256,), _I),
          pltpu.VMEM_SHARED((NB * L,), _I),
          pltpu.VMEM_SHARED((NS * NB * L,), _F),
      ],
  )
  return f(aux, conf_p)


def kernel(loc_data, conf_data, priors, targets):
  aux = jnp.concatenate([
      jnp.pad(loc_data, ((0, 0), (0, 8960 - P), (0, 0))).reshape(-1),
      jnp.pad(priors, ((0, 8960 - P), (0, 0))).reshape(-1),
      targets.reshape(-1)])
  out = _run(aux, conf_data)
  loss_l = out[0] + out[L]
  loss_c = out[1] + out[L + 1]
  n = out[2] + out[L + 2]
  return loss_l / n, loss_c / n


# speedup vs baseline: 1.0043x; 1.0043x over previous
import jax
import jax.numpy as jnp
from jax import lax
from jax.experimental import pallas as pl
from jax.experimental.pallas import tpu as pltpu, tpu_sc as plsc

N_CLASSES = 21
THRESHOLD = 0.5
NEGPOS_RATIO = 3
VAR0 = 0.1
VAR1 = 0.2

B = 8
P = 8732
O = 16
NC = 2
NS = 16
L = 16
NB = B // NC
VPS = 560
OWN = 552
DMAR = 560
TSTART = 8724
NV = VPS // L
C = N_CLASSES

_F = jnp.float32
_I = jnp.int32


def _iota():
  return lax.iota(_I, L)


def _spl_f(x):
  return jnp.full((L,), x, _F)


def _spl_i(x):
  return jnp.full((L,), x, _I)


def _vln(x):
  x = jnp.maximum(x, _spl_f(1e-30))
  bits = plsc.bitcast(x, _I)
  e = lax.shift_right_logical(bits, _spl_i(23)) - _spl_i(127)
  m = plsc.bitcast((bits & _spl_i(0x007FFFFF)) | _spl_i(127 << 23), _F)
  big = m > _spl_f(1.4142135623730951)
  m = jnp.where(big, m * _spl_f(0.5), m)
  e = e + jnp.where(big, _spl_i(1), _spl_i(0))
  z = (m - _spl_f(1.0)) / (m + _spl_f(1.0))
  z2 = z * z
  p = _spl_f(1.0) + z2 * (_spl_f(1.0 / 3) + z2 * (_spl_f(1.0 / 5)
                                                  + z2 * _spl_f(1.0 / 7)))
  return _spl_f(2.0) * z * p + e.astype(_F) * _spl_f(0.6931471805599453)


def _lane_extract_i(vec, lane_splat):
  return jnp.max(jnp.where(_iota() == lane_splat, vec, _spl_i(-2**31 + 1)))


LOFF = 0
POFF = B * 8960 * 4
TOFF = POFF + 8960 * 4


def _sc_body(aux_hbm, conf_hbm, out_hbm,
             conf_v, loc_v, pri_v, tgt_v, trd_v, bto_v, bti_v,
             amv_v, ami_v, msk_v, hist_v, red_v, totf_v, sta_v, fin_v,
             bpi_v, thr_v, scr_v, scri_v,
             amv_s, ami_s, bpi_s, sta_s, his_s, thr_s, fin_s):
  cid = lax.axis_index("c")
  sid = lax.axis_index("s")
  p0 = sid * OWN
  start = pl.multiple_of(jnp.minimum(p0, 8176), 8)
  iot = _iota()
  p0v = jnp.full((L,), p0, _I)
  dlv = jnp.full((L,), p0 - start, _I)
  cidv = jnp.full((L,), cid, _I)

  def prow(pvec, gidx):
    return jnp.minimum(pvec + dlv, _spl_i(555))

  def spl(x, dtype=_I):
    return jnp.full((L,), x, dtype)

  def gat(ref, idx):
    return plsc.load_gather(ref, [idx])

  def sca(ref, idx, val, mask=None):
    plsc.store_scatter(ref, [idx], val, mask=mask)

  pltpu.sync_copy(aux_hbm.at[pl.ds(POFF + p0 * 4, VPS * 4)], pri_v)
  pltpu.sync_copy(aux_hbm.at[pl.ds(TOFF + cid * NB * O * 5, NB * O * 5)],
                  tgt_v)

  def truthd_body(b, _):
    tb = spl(b * O * 5) + iot * _spl_i(5)
    tx1 = gat(tgt_v, tb)
    ty1 = gat(tgt_v, tb + _spl_i(1))
    tx2 = gat(tgt_v, tb + _spl_i(2))
    ty2 = gat(tgt_v, tb + _spl_i(3))
    lab = gat(tgt_v, tb + _spl_i(4))
    dst = spl(b * 8 * L) + iot
    sca(trd_v, dst, tx1)
    sca(trd_v, dst + _spl_i(L), ty1)
    sca(trd_v, dst + _spl_i(2 * L), tx2)
    sca(trd_v, dst + _spl_i(3 * L), ty2)
    sca(trd_v, dst + _spl_i(4 * L), (tx2 - tx1) * (ty2 - ty1))
    sca(trd_v, dst + _spl_i(5 * L), lab)
    return 0
  lax.fori_loop(0, NB, truthd_body, 0)

  def jac_b(b, _):
    tbase = spl(b * 8 * L)
    abase = spl(b * O * L)

    def init_o(o, _):
      dst = abase + spl(o * L) + iot
      sca(amv_v, dst, _spl_f(-1.0))
      sca(ami_v, dst, _spl_f(0.0))
      return 0
    lax.fori_loop(0, O, init_o, 0)

    def jac_v(v, _):
      pvec = v * L + iot
      gidx = p0v + pvec
      gidxf = gidx.astype(_F)
      validv = (pvec < _spl_i(OWN)) & (gidx < _spl_i(P))
      row = prow(pvec, gidx)
      p4 = pvec * _spl_i(4)
      pcx = gat(pri_v, p4)
      pcy = gat(pri_v, p4 + _spl_i(1))
      pw = gat(pri_v, p4 + _spl_i(2))
      ph = gat(pri_v, p4 + _spl_i(3))
      pw2 = pw * _spl_f(0.5)
      ph2 = ph * _spl_f(0.5)
      px1 = pcx - pw2
      px2 = pcx + pw2
      py1 = pcy - ph2
      py2 = pcy + ph2
      areab = pw * ph
      bto = _spl_f(-1.0)
      bti = _spl_f(0.0)
      for o in range(O):
        tx1 = gat(trd_v, tbase + _spl_i(o))
        ty1 = gat(trd_v, tbase + _spl_i(L + o))
        tx2 = gat(trd_v, tbase + _spl_i(2 * L + o))
        ty2 = gat(trd_v, tbase + _spl_i(3 * L + o))
        ta = gat(trd_v, tbase + _spl_i(4 * L + o))
        iw = jnp.maximum(jnp.minimum(px2, tx2) - jnp.maximum(px1, tx1),
                         _spl_f(0.0))
        ih = jnp.maximum(jnp.minimum(py2, ty2) - jnp.maximum(py1, ty1),
                         _spl_f(0.0))
        inter = iw * ih
        ov = inter / (ta + areab - inter)
        m = ov > bto
        bto = jnp.where(m, ov, bto)
        bti = jnp.where(m, _spl_f(o), bti)
        adst = abase + spl(o * L) + iot
        av = gat(amv_v, adst)
        ai = gat(ami_v, adst)
        mu = (ov > av) & validv
        sca(amv_v, adst, jnp.where(mu, ov, av))
        sca(ami_v, adst, jnp.where(mu, gidxf, ai))
      bdst = spl(b * VPS) + pvec
      sca(bto_v, bdst, bto)
      sca(bti_v, bdst, bti)
      return 0
    lax.fori_loop(0, NV, jac_v, 0)
    return 0
  lax.fori_loop(0, NB, jac_b, 0)

  def pub_am(b, _):
    pltpu.sync_copy(amv_v.at[pl.ds(b * O * L, O * L)],
                    amv_s.at[pl.ds(b * NS * O * L + sid * O * L, O * L)])
    pltpu.sync_copy(ami_v.at[pl.ds(b * O * L, O * L)],
                    ami_s.at[pl.ds(b * NS * O * L + sid * O * L, O * L)])
    return 0
  lax.fori_loop(0, NB, pub_am, 0)
  plsc.subcore_barrier()

  @pl.when(sid < NB)
  def _():
    jb = sid
    pltpu.sync_copy(amv_s.at[pl.ds(jb * NS * O * L, NS * O * L)], amv_v)
    pltpu.sync_copy(ami_s.at[pl.ds(jb * NS * O * L, NS * O * L)], ami_v)
    bvec = _spl_f(0.0)
    for o in range(O):
      def red_s(s2, carry):
        rv, ri = carry
        src = spl(s2 * O * L + o * L) + iot
        av = gat(amv_v, src)
        ai = gat(ami_v, src)
        better = (av > rv) | ((av == rv) & (ai < ri))
        return jnp.where(better, av, rv), jnp.where(better, ai, ri)
      rv, ri = lax.fori_loop(0, NS, red_s, (_spl_f(-1.0), _spl_f(0.0)))
      mx = jnp.max(rv)
      cand = jnp.where(rv == jnp.full((L,), mx, _F), ri, _spl_f(3e38))
      bpio = jnp.min(cand)
      bvec = jnp.where(iot == _spl_i(o), jnp.full((L,), bpio, _F), bvec)
    sca(scr_v, iot, bvec)
    pltpu.sync_copy(scr_v.at[pl.ds(0, L)], bpi_s.at[pl.ds(jb * L, L)])
  plsc.subcore_barrier()

  pltpu.sync_copy(bpi_s, bpi_v)

  def ph2_b(b, carry):
    accl_t, accce_t = carry
    gb = cid * NB + b
    pltpu.sync_copy(conf_hbm.at[gb, pl.ds(start, DMAR), :], conf_v)
    pltpu.sync_copy(aux_hbm.at[pl.ds(gb * 35840 + p0 * 4, VPS * 4)], loc_v)
    tbase = spl(b * 8 * L)
    bbase = spl(b * VPS)
    bv = spl(b)
    bpirow = gat(bpi_v, spl(b * L) + iot).astype(_I)
    localp = bpirow - p0v
    inr = (localp >= _spl_i(0)) & (localp < _spl_i(OWN))
    localc = jnp.minimum(jnp.maximum(localp, _spl_i(0)),
                         _spl_i(VPS - 1)) + bbase
    for o in range(O):
      mo = inr & (iot == _spl_i(o))
      sca(bto_v, localc, _spl_f(2.0), mask=mo)
      sca(bti_v, localc, _spl_f(o), mask=mo)

    def ph2_v(v, carry):
      accl, accce, accnp = carry
      pvec = v * L + iot
      gidx = p0v + pvec
      bto = gat(bto_v, bbase + pvec)
      bti = gat(bti_v, bbase + pvec)
      pos = bto >= _spl_f(THRESHOLD)
      valid = (pvec < _spl_i(OWN)) & (gidx < _spl_i(P))
      row = prow(pvec, gidx)
      btii = bti.astype(_I)
      mx1 = gat(trd_v, tbase + btii)
      my1 = gat(trd_v, tbase + _spl_i(L) + btii)
      mx2 = gat(trd_v, tbase + _spl_i(2 * L) + btii)
      my2 = gat(trd_v, tbase + _spl_i(3 * L) + btii)
      p4 = pvec * _spl_i(4)
      pcx = gat(pri_v, p4)
      pcy = gat(pri_v, p4 + _spl_i(1))
      pw = gat(pri_v, p4 + _spl_i(2))
      ph = gat(pri_v, p4 + _spl_i(3))
      gx = ((mx1 + mx2) * _spl_f(0.5) - pcx) / (pw * _spl_f(VAR0))
      gy = ((my1 + my2) * _spl_f(0.5) - pcy) / (ph * _spl_f(VAR0))
      gw = _vln((mx2 - mx1) / pw) * _spl_f(1.0 / VAR1)
      gh = _vln((my2 - my1) / ph) * _spl_f(1.0 / VAR1)
      sl = _spl_f(0.0)
      for c4, g in ((0, gx), (1, gy), (2, gw), (3, gh)):
        lc = gat(loc_v, p4 + _spl_i(c4))
        d = lc - g
        ad = jnp.abs(d)
        sl = sl + jnp.where(ad < _spl_f(1.0), _spl_f(0.5) * d * d,
                            ad - _spl_f(0.5))
      cls_vals = [plsc.load_gather(conf_v, [row, _spl_i(c)])
                  for c in range(C)]
      mrow = cls_vals[0]
      for cv in cls_vals[1:]:
        mrow = jnp.maximum(mrow, cv)
      ssum = _spl_f(0.0)
      for cv in cls_vals:
        ssum = ssum + jnp.exp(cv - mrow)
      lse = _vln(ssum) + mrow
      cls = jnp.where(pos, _spl_i(1), _spl_i(0))
      quirk = (gidx == _spl_i(0)) & (cidv == _spl_i(0)) & \
          (bv == _spl_i(0))
      cls = jnp.where(quirk, _spl_i(1), cls)
      g0 = jnp.where(cls == _spl_i(1), cls_vals[1], cls_vals[0])
      ce = lse - g0
      posv = pos & valid
      accl = accl + jnp.where(posv, sl, _spl_f(0.0))
      accce = accce + jnp.where(posv, ce, _spl_f(0.0))
      accnp = accnp + jnp.where(posv, _spl_f(1.0), _spl_f(0.0))
      sca(msk_v, bbase + pvec,
          jnp.where(valid & jnp.logical_not(pos),
                    jnp.maximum(ce, _spl_f(0.0)), _spl_f(0.0)))
      return accl, accce, accnp

    accl, accce, accnp = lax.fori_loop(
        0, NV, ph2_v, (_spl_f(0.0), _spl_f(0.0), _spl_f(0.0)))
    sca(scr_v, spl(b * L) + iot, accnp)
    return accl_t + accl, accce_t + accce

  accl_t, accce_t = lax.fori_loop(0, NB, ph2_b, (_spl_f(0.0), _spl_f(0.0)))
  sca(scr_v, spl(NB * L) + iot, accl_t)
  sca(scr_v, spl((NB + 1) * L) + iot, accce_t)
  pltpu.sync_copy(scr_v.at[pl.ds(0, 6 * L)], sta_s.at[pl.ds(sid * 6 * L,
                                                            6 * L)])

  ones = _spl_i(1)

  def round_r(r, _):
    shift = _spl_i(24) - r * _spl_i(8)
    shift_hi = jnp.minimum(shift + _spl_i(8), _spl_i(31))
    is0v = jnp.full((L,), r == 0, jnp.bool_)

    def hist_b(b, _):
      hbase = spl(b * 256)
      def zero_h(i, _):
        sca(hist_v, hbase + spl(i * L) + iot, _spl_i(0))
        return 0
      lax.fori_loop(0, 256 // L, zero_h, 0)
      pref = gat(thr_v, spl(b * L))
      bbase = spl(b * VPS)
      def hist_vv(v, _):
        mvec = gat(msk_v, bbase + v * L + iot)
        bits = plsc.bitcast(mvec, _I)
        match = is0v | (lax.shift_right_logical(bits, shift_hi) == pref)
        bucket = lax.shift_right_logical(bits, shift) & _spl_i(255)
        plsc.addupdate_scatter(hist_v, [hbase + bucket], ones, mask=match)
        return 0
      lax.fori_loop(0, NV, hist_vv, 0)
      pltpu.sync_copy(hist_v.at[pl.ds(b * 256, 256)],
                      his_s.at[pl.ds(b * NS * 256 + sid * 256, 256)])
      return 0
    lax.fori_loop(0, NB, hist_b, 0)
    plsc.subcore_barrier()

    @pl.when(sid < NB)
    def _():
      jb = sid
      jbv = jnp.full((L,), jb, _I)
      pltpu.sync_copy(his_s.at[pl.ds(jb * NS * 256, NS * 256)], red_v)
      pltpu.sync_copy(sta_s, sta_v)
      def nps(s2, acc):
        return acc + gat(sta_v, spl(s2 * 6 * L + jb * L) + iot)
      npsum = lax.fori_loop(0, NS, nps, _spl_f(0.0))
      npb = jnp.sum(npsum)
      k = jnp.minimum(npb * jnp.float32(NEGPOS_RATIO), jnp.float32(P - 1))
      ki = k.astype(_I)
      pref_old = gat(thr_v, jbv * _spl_i(L))
      kk_old = gat(thr_v, jbv * _spl_i(L) + _spl_i(1))
      kkv = jnp.where(is0v, jnp.full((L,), ki, _I), kk_old)
      def tot_i(i, _):
        def tadd(s2, acc):
          return acc + gat(red_v, spl(s2 * 256 + i * L) + iot)
        t = lax.fori_loop(0, NS, tadd, _spl_i(0))
        sca(totf_v, spl(i * L) + iot, t)
        return 0
      lax.fori_loop(0, 256 // L, tot_i, 0)
      def chunk(ic, carry):
        crun, found, gsel, kksel = carry
        i = _spl_i(255 // L) - ic
        tvec = gat(totf_v, i * _spl_i(L) + iot)
        rev = lax.rev(tvec, (0,))
        cs = plsc.cumsum(rev)
        tot = jnp.full((L,), jnp.max(cs), _I)
        candv = crun + cs
        hitm = candv >= kkv
        anyhit = jnp.full((L,), jnp.max(hitm.astype(_I)), _I) > _spl_i(0)
        ffs = plsc.all_reduce_ffs(hitm)
        gcand = i * _spl_i(L) + _spl_i(L - 1) - ffs
        cnt_at = jnp.full((L,), _lane_extract_i(rev, ffs), _I)
        cum_at = jnp.full((L,), _lane_extract_i(candv, ffs), _I)
        kkcand = kkv - (cum_at - cnt_at)
        take = jnp.logical_not(found) & anyhit
        gsel = jnp.where(take, gcand, gsel)
        kksel = jnp.where(take, kkcand, kksel)
        found = found | anyhit
        crun = crun + tot
        return crun, found, gsel, kksel
      crun, found, gsel, kksel = lax.fori_loop(
          0, 256 // L, chunk,
          (_spl_i(0), jnp.full((L,), False, jnp.bool_), _spl_i(0),
           _spl_i(0)))
      pnew = jnp.where(is0v, gsel,
                       lax.shift_left(pref_old, _spl_i(8)) | gsel)
      outv = jnp.where(iot == _spl_i(0), pnew,
                       jnp.where(iot == _spl_i(1), kksel, _spl_i(0)))
      sca(scri_v, iot, outv)
      pltpu.sync_copy(scri_v.at[pl.ds(0, L)], thr_s.at[pl.ds(jb * L, L)])
    plsc.subcore_barrier()
    pltpu.sync_copy(thr_s, thr_v)
    return 0

  def zthr(b, _):
    sca(thr_v, spl(b * L) + iot, _spl_i(0))
    return 0
  lax.fori_loop(0, NB, zthr, 0)
  lax.fori_loop(0, 4, round_r, 0)

  def fin_b(b, _):
    tbits = gat(thr_v, spl(b * L))
    tval = plsc.bitcast(tbits, _F)
    bbase = spl(b * VPS)
    def fin_vv(v, acc):
      mvec = gat(msk_v, bbase + v * L + iot)
      return acc + jnp.where(mvec > tval, mvec, _spl_f(0.0))
    acc = lax.fori_loop(0, NV, fin_vv, _spl_f(0.0))
    sca(scr_v, spl(b * L) + iot, acc)
    return 0
  lax.fori_loop(0, NB, fin_b, 0)
  pltpu.sync_copy(scr_v.at[pl.ds(0, NB * L)],
                  fin_s.at[pl.ds(sid * NB * L, NB * L)])
  plsc.subcore_barrier()

  @pl.when(sid == 0)
  def _():
    pltpu.sync_copy(fin_s, fin_v)
    pltpu.sync_copy(sta_s, sta_v)
    def sacc(s2, carry):
      lsum, cesum = carry
      return (lsum + gat(sta_v, spl(s2 * 6 * L + NB * L) + iot),
              cesum + gat(sta_v, spl(s2 * 6 * L + (NB + 1) * L) + iot))
    lsum, cesum = lax.fori_loop(0, NS, sacc, (_spl_f(0.0), _spl_f(0.0)))
    loss_l = jnp.sum(lsum)
    loss_c = jnp.sum(cesum)
    nsum = jnp.float32(0.0)
    for jb in range(NB):
      def npacc(s2, acc):
        return acc + gat(sta_v, spl(s2 * 6 * L + jb * L) + iot)
      npv = lax.fori_loop(0, NS, npacc, _spl_f(0.0))
      nsum = nsum + jnp.sum(npv)
      def facc(s2, acc):
        return acc + gat(fin_v, spl(s2 * NB * L + jb * L) + iot)
      sv = lax.fori_loop(0, NS, facc, _spl_f(0.0))
      tbits = gat(thr_v, _spl_i(jb * L))
      kkrem = gat(thr_v, _spl_i(jb * L + 1))
      tval = plsc.bitcast(tbits, _F)
      loss_c = loss_c + jnp.sum(sv) + jnp.max(kkrem).astype(_F) \
          * jnp.max(tval)
    ovec = jnp.where(iot == _spl_i(0), jnp.full((L,), loss_l, _F),
                     jnp.where(iot == _spl_i(1), jnp.full((L,), loss_c, _F),
                               jnp.where(iot == _spl_i(2),
                                         jnp.full((L,), nsum, _F),
                                         _spl_f(0.0))))
    sca(scr_v, iot, ovec)
    pltpu.sync_copy(scr_v.at[pl.ds(0, L)], out_hbm.at[pl.ds(cid * L, L)])


@jax.jit
def _run(aux, conf_p):
  mesh = plsc.VectorSubcoreMesh(core_axis_name="c", subcore_axis_name="s")
  f = pl.kernel(
      _sc_body,
      out_type=jax.ShapeDtypeStruct((NC * L,), _F),
      mesh=mesh,
      compiler_params=pltpu.CompilerParams(needs_layout_passes=False),
      scratch_types=[
          pltpu.VMEM((DMAR, C), _F),
          pltpu.VMEM((VPS * 4,), _F),
          pltpu.VMEM((VPS * 4,), _F),
          pltpu.VMEM((NB * O * 5,), _F),
          pltpu.VMEM((NB * 8 * L,), _F),
          pltpu.VMEM((NB * VPS,), _F),
          pltpu.VMEM((NB * VPS,), _F),
          pltpu.VMEM((NS * O * L,), _F),
          pltpu.VMEM((NS * O * L,), _F),
          pltpu.VMEM((NB * VPS,), _F),
          pltpu.VMEM((NB * 256,), _I),
          pltpu.VMEM((NS * 256,), _I),
          pltpu.VMEM((256,), _I),
          pltpu.VMEM((NS * 6 * L,), _F),
          pltpu.VMEM((NS * NB * L,), _F),
          pltpu.VMEM((NB * L,), _F),
          pltpu.VMEM((NB * L,), _I),
          pltpu.VMEM((8 * L,), _F),
          pltpu.VMEM((2 * L,), _I),
          pltpu.VMEM_SHARED((NB * NS * O * L,), _F),
          pltpu.VMEM_SHARED((NB * NS * O * L,), _F),
          pltpu.VMEM_SHARED((NB * L,), _F),
          pltpu.VMEM_SHARED((NS * 6 * L,), _F),
          pltpu.VMEM_SHARED((NB * NS * 256,), _I),
          pltpu.VMEM_SHARED((NB * L,), _I),
          pltpu.VMEM_SHARED((NS * NB * L,), _F),
      ],
  )
  return f(aux, conf_p)


def kernel(loc_data, conf_data, priors, targets):
  aux = jnp.concatenate([
      jnp.pad(loc_data, ((0, 0), (0, 8960 - P), (0, 0))).reshape(-1),
      jnp.pad(priors, ((0, 8960 - P), (0, 0))).reshape(-1),
      targets.reshape(-1)])
  out = _run(aux, conf_data)
  loss_l = out[0] + out[L]
  loss_c = out[1] + out[L + 1]
  n = out[2] + out[L + 2]
  return loss_l / n, loss_c / n


# speedup vs baseline: 1.0743x; 1.0697x over previous
import jax
import jax.numpy as jnp
from jax import lax
from jax.experimental import pallas as pl
from jax.experimental.pallas import tpu as pltpu, tpu_sc as plsc

N_CLASSES = 21
THRESHOLD = 0.5
NEGPOS_RATIO = 3
VAR0 = 0.1
VAR1 = 0.2

B = 8
P = 8732
O = 16
NC = 2
NS = 16
L = 16
NB = B // NC
VPS = 560
OWN = 552
DMAR = 560
TSTART = 8724
NV = VPS // L
C = N_CLASSES

_F = jnp.float32
_I = jnp.int32


def _iota():
  return lax.iota(_I, L)


def _spl_f(x):
  return jnp.full((L,), x, _F)


def _spl_i(x):
  return jnp.full((L,), x, _I)


def _vln(x):
  x = jnp.maximum(x, _spl_f(1e-30))
  bits = plsc.bitcast(x, _I)
  e = lax.shift_right_logical(bits, _spl_i(23)) - _spl_i(127)
  m = plsc.bitcast((bits & _spl_i(0x007FFFFF)) | _spl_i(127 << 23), _F)
  big = m > _spl_f(1.4142135623730951)
  m = jnp.where(big, m * _spl_f(0.5), m)
  e = e + jnp.where(big, _spl_i(1), _spl_i(0))
  z = (m - _spl_f(1.0)) / (m + _spl_f(1.0))
  z2 = z * z
  p = _spl_f(1.0) + z2 * (_spl_f(1.0 / 3) + z2 * (_spl_f(1.0 / 5)
                                                  + z2 * _spl_f(1.0 / 7)))
  return _spl_f(2.0) * z * p + e.astype(_F) * _spl_f(0.6931471805599453)


def _lane_extract_i(vec, lane_splat):
  return jnp.max(jnp.where(_iota() == lane_splat, vec, _spl_i(-2**31 + 1)))


LOFF = 0
POFF = B * 8960 * 4
TOFF = POFF + 8960 * 4


def _sc_body(aux_hbm, conf_hbm, out_hbm,
             conf_v, loc_v, pri_v, tgt_v, trd_v, bto_v, bti_v,
             amv_v, ami_v, msk_v, hist_v, red_v, totf_v, sta_v, fin_v,
             bpi_v, thr_v, scr_v, scri_v,
             amv_s, ami_s, bpi_s, sta_s, his_s, thr_s, fin_s):
  cid = lax.axis_index("c")
  sid = lax.axis_index("s")
  p0 = sid * OWN
  start = pl.multiple_of(jnp.minimum(p0, 8176), 8)
  iot = _iota()
  p0v = jnp.full((L,), p0, _I)
  dlv = jnp.full((L,), p0 - start, _I)
  cidv = jnp.full((L,), cid, _I)

  def prow(pvec, gidx):
    return jnp.minimum(pvec + dlv, _spl_i(555))

  def spl(x, dtype=_I):
    return jnp.full((L,), x, dtype)

  def gat(ref, idx):
    return plsc.load_gather(ref, [idx])

  def sca(ref, idx, val, mask=None):
    plsc.store_scatter(ref, [idx], val, mask=mask)

  pltpu.sync_copy(aux_hbm.at[pl.ds(POFF + p0 * 4, VPS * 4)], pri_v)
  pltpu.sync_copy(aux_hbm.at[pl.ds(TOFF + cid * NB * O * 5, NB * O * 5)],
                  tgt_v)

  def truthd_body(b, _):
    tb = spl(b * O * 5) + iot * _spl_i(5)
    tx1 = gat(tgt_v, tb)
    ty1 = gat(tgt_v, tb + _spl_i(1))
    tx2 = gat(tgt_v, tb + _spl_i(2))
    ty2 = gat(tgt_v, tb + _spl_i(3))
    lab = gat(tgt_v, tb + _spl_i(4))
    dst = spl(b * 8 * L) + iot
    sca(trd_v, dst, tx1)
    sca(trd_v, dst + _spl_i(L), ty1)
    sca(trd_v, dst + _spl_i(2 * L), tx2)
    sca(trd_v, dst + _spl_i(3 * L), ty2)
    sca(trd_v, dst + _spl_i(4 * L), (tx2 - tx1) * (ty2 - ty1))
    sca(trd_v, dst + _spl_i(5 * L), lab)
    return 0
  lax.fori_loop(0, NB, truthd_body, 0)

  def jac_b(b, _):
    tbase = spl(b * 8 * L)
    abase = spl(b * O * L)
    HO = O // 2

    for half in range(2):
      def jac_v(v, carry):
        avs = list(carry[:HO])
        ais = list(carry[HO:])
        pvec = v * L + iot
        gidx = p0v + pvec
        gidxf = gidx.astype(_F)
        validv = (pvec < _spl_i(OWN)) & (gidx < _spl_i(P))
        p4 = pvec * _spl_i(4)
        pcx = gat(pri_v, p4)
        pcy = gat(pri_v, p4 + _spl_i(1))
        pw = gat(pri_v, p4 + _spl_i(2))
        ph = gat(pri_v, p4 + _spl_i(3))
        pw2 = pw * _spl_f(0.5)
        ph2 = ph * _spl_f(0.5)
        px1 = pcx - pw2
        px2 = pcx + pw2
        py1 = pcy - ph2
        py2 = pcy + ph2
        areab = pw * ph
        bdst = spl(b * VPS) + pvec
        if half == 0:
          bto = _spl_f(-1.0)
          bti = _spl_f(0.0)
        else:
          bto = gat(bto_v, bdst)
          bti = gat(bti_v, bdst)
        for k in range(HO):
          o = half * HO + k
          tx1 = gat(trd_v, tbase + _spl_i(o))
          ty1 = gat(trd_v, tbase + _spl_i(L + o))
          tx2 = gat(trd_v, tbase + _spl_i(2 * L + o))
          ty2 = gat(trd_v, tbase + _spl_i(3 * L + o))
          ta = gat(trd_v, tbase + _spl_i(4 * L + o))
          iw = jnp.maximum(jnp.minimum(px2, tx2) - jnp.maximum(px1, tx1),
                           _spl_f(0.0))
          ih = jnp.maximum(jnp.minimum(py2, ty2) - jnp.maximum(py1, ty1),
                           _spl_f(0.0))
          inter = iw * ih
          ov = inter / (ta + areab - inter)
          m = ov > bto
          bto = jnp.where(m, ov, bto)
          bti = jnp.where(m, _spl_f(o), bti)
          mu = (ov > avs[k]) & validv
          avs[k] = jnp.where(mu, ov, avs[k])
          ais[k] = jnp.where(mu, gidxf, ais[k])
        sca(bto_v, bdst, bto)
        sca(bti_v, bdst, bti)
        return tuple(avs) + tuple(ais)

      init = tuple(_spl_f(-1.0) for _ in range(HO)) \
          + tuple(_spl_f(0.0) for _ in range(HO))
      fin = lax.fori_loop(0, NV, jac_v, init)
      for k in range(HO):
        o = half * HO + k
        dst = abase + spl(o * L) + iot
        sca(amv_v, dst, fin[k])
        sca(ami_v, dst, fin[HO + k])
    return 0
  lax.fori_loop(0, NB, jac_b, 0)

  def pub_am(b, _):
    pltpu.sync_copy(amv_v.at[pl.ds(b * O * L, O * L)],
                    amv_s.at[pl.ds(b * NS * O * L + sid * O * L, O * L)])
    pltpu.sync_copy(ami_v.at[pl.ds(b * O * L, O * L)],
                    ami_s.at[pl.ds(b * NS * O * L + sid * O * L, O * L)])
    return 0
  lax.fori_loop(0, NB, pub_am, 0)
  plsc.subcore_barrier()

  @pl.when(sid < NB)
  def _():
    jb = sid
    pltpu.sync_copy(amv_s.at[pl.ds(jb * NS * O * L, NS * O * L)], amv_v)
    pltpu.sync_copy(ami_s.at[pl.ds(jb * NS * O * L, NS * O * L)], ami_v)
    bvec = _spl_f(0.0)
    for o in range(O):
      def red_s(s2, carry):
        rv, ri = carry
        src = spl(s2 * O * L + o * L) + iot
        av = gat(amv_v, src)
        ai = gat(ami_v, src)
        better = (av > rv) | ((av == rv) & (ai < ri))
        return jnp.where(better, av, rv), jnp.where(better, ai, ri)
      rv, ri = lax.fori_loop(0, NS, red_s, (_spl_f(-1.0), _spl_f(0.0)))
      mx = jnp.max(rv)
      cand = jnp.where(rv == jnp.full((L,), mx, _F), ri, _spl_f(3e38))
      bpio = jnp.min(cand)
      bvec = jnp.where(iot == _spl_i(o), jnp.full((L,), bpio, _F), bvec)
    sca(scr_v, iot, bvec)
    pltpu.sync_copy(scr_v.at[pl.ds(0, L)], bpi_s.at[pl.ds(jb * L, L)])
  plsc.subcore_barrier()

  pltpu.sync_copy(bpi_s, bpi_v)

  def ph2_b(b, carry):
    accl_t, accce_t = carry
    gb = cid * NB + b
    pltpu.sync_copy(conf_hbm.at[gb, pl.ds(start, DMAR), :], conf_v)
    pltpu.sync_copy(aux_hbm.at[pl.ds(gb * 35840 + p0 * 4, VPS * 4)], loc_v)
    tbase = spl(b * 8 * L)
    bbase = spl(b * VPS)
    bv = spl(b)
    bpirow = gat(bpi_v, spl(b * L) + iot).astype(_I)
    localp = bpirow - p0v
    inr = (localp >= _spl_i(0)) & (localp < _spl_i(OWN))
    localc = jnp.minimum(jnp.maximum(localp, _spl_i(0)),
                         _spl_i(VPS - 1)) + bbase
    for o in range(O):
      mo = inr & (iot == _spl_i(o))
      sca(bto_v, localc, _spl_f(2.0), mask=mo)
      sca(bti_v, localc, _spl_f(o), mask=mo)

    def ph2_v(v, carry):
      accl, accce, accnp = carry
      pvec = v * L + iot
      gidx = p0v + pvec
      bto = gat(bto_v, bbase + pvec)
      bti = gat(bti_v, bbase + pvec)
      pos = bto >= _spl_f(THRESHOLD)
      valid = (pvec < _spl_i(OWN)) & (gidx < _spl_i(P))
      row = prow(pvec, gidx)
      btii = bti.astype(_I)
      mx1 = gat(trd_v, tbase + btii)
      my1 = gat(trd_v, tbase + _spl_i(L) + btii)
      mx2 = gat(trd_v, tbase + _spl_i(2 * L) + btii)
      my2 = gat(trd_v, tbase + _spl_i(3 * L) + btii)
      p4 = pvec * _spl_i(4)
      pcx = gat(pri_v, p4)
      pcy = gat(pri_v, p4 + _spl_i(1))
      pw = gat(pri_v, p4 + _spl_i(2))
      ph = gat(pri_v, p4 + _spl_i(3))
      gx = ((mx1 + mx2) * _spl_f(0.5) - pcx) / (pw * _spl_f(VAR0))
      gy = ((my1 + my2) * _spl_f(0.5) - pcy) / (ph * _spl_f(VAR0))
      gw = _vln((mx2 - mx1) / pw) * _spl_f(1.0 / VAR1)
      gh = _vln((my2 - my1) / ph) * _spl_f(1.0 / VAR1)
      sl = _spl_f(0.0)
      for c4, g in ((0, gx), (1, gy), (2, gw), (3, gh)):
        lc = gat(loc_v, p4 + _spl_i(c4))
        d = lc - g
        ad = jnp.abs(d)
        sl = sl + jnp.where(ad < _spl_f(1.0), _spl_f(0.5) * d * d,
                            ad - _spl_f(0.5))
      cls_vals = [plsc.load_gather(conf_v, [row, _spl_i(c)])
                  for c in range(C)]
      mrow = cls_vals[0]
      for cv in cls_vals[1:]:
        mrow = jnp.maximum(mrow, cv)
      ssum = _spl_f(0.0)
      for cv in cls_vals:
        ssum = ssum + jnp.exp(cv - mrow)
      lse = _vln(ssum) + mrow
      cls = jnp.where(pos, _spl_i(1), _spl_i(0))
      quirk = (gidx == _spl_i(0)) & (cidv == _spl_i(0)) & \
          (bv == _spl_i(0))
      cls = jnp.where(quirk, _spl_i(1), cls)
      g0 = jnp.where(cls == _spl_i(1), cls_vals[1], cls_vals[0])
      ce = lse - g0
      posv = pos & valid
      accl = accl + jnp.where(posv, sl, _spl_f(0.0))
      accce = accce + jnp.where(posv, ce, _spl_f(0.0))
      accnp = accnp + jnp.where(posv, _spl_f(1.0), _spl_f(0.0))
      sca(msk_v, bbase + pvec,
          jnp.where(valid & jnp.logical_not(pos),
                    jnp.maximum(ce, _spl_f(0.0)), _spl_f(0.0)))
      return accl, accce, accnp

    accl, accce, accnp = lax.fori_loop(
        0, NV, ph2_v, (_spl_f(0.0), _spl_f(0.0), _spl_f(0.0)))
    sca(scr_v, spl(b * L) + iot, accnp)
    return accl_t + accl, accce_t + accce

  accl_t, accce_t = lax.fori_loop(0, NB, ph2_b, (_spl_f(0.0), _spl_f(0.0)))
  sca(scr_v, spl(NB * L) + iot, accl_t)
  sca(scr_v, spl((NB + 1) * L) + iot, accce_t)
  pltpu.sync_copy(scr_v.at[pl.ds(0, 6 * L)], sta_s.at[pl.ds(sid * 6 * L,
                                                            6 * L)])

  ones = _spl_i(1)

  def round_r(r, _):
    shift = _spl_i(24) - r * _spl_i(8)
    shift_hi = jnp.minimum(shift + _spl_i(8), _spl_i(31))
    is0v = jnp.full((L,), r == 0, jnp.bool_)

    def hist_b(b, _):
      hbase = spl(b * 256)
      def zero_h(i, _):
        sca(hist_v, hbase + spl(i * L) + iot, _spl_i(0))
        return 0
      lax.fori_loop(0, 256 // L, zero_h, 0)
      pref = gat(thr_v, spl(b * L))
      bbase = spl(b * VPS)
      def hist_vv(v, _):
        mvec = gat(msk_v, bbase + v * L + iot)
        bits = plsc.bitcast(mvec, _I)
        match = is0v | (lax.shift_right_logical(bits, shift_hi) == pref)
        bucket = lax.shift_right_logical(bits, shift) & _spl_i(255)
        plsc.addupdate_scatter(hist_v, [hbase + bucket], ones, mask=match)
        return 0
      lax.fori_loop(0, NV, hist_vv, 0)
      pltpu.sync_copy(hist_v.at[pl.ds(b * 256, 256)],
                      his_s.at[pl.ds(b * NS * 256 + sid * 256, 256)])
      return 0
    lax.fori_loop(0, NB, hist_b, 0)
    plsc.subcore_barrier()

    @pl.when(sid < NB)
    def _():
      jb = sid
      jbv = jnp.full((L,), jb, _I)
      pltpu.sync_copy(his_s.at[pl.ds(jb * NS * 256, NS * 256)], red_v)
      pltpu.sync_copy(sta_s, sta_v)
      def nps(s2, acc):
        return acc + gat(sta_v, spl(s2 * 6 * L + jb * L) + iot)
      npsum = lax.fori_loop(0, NS, nps, _spl_f(0.0))
      npb = jnp.sum(npsum)
      k = jnp.minimum(npb * jnp.float32(NEGPOS_RATIO), jnp.float32(P - 1))
      ki = k.astype(_I)
      pref_old = gat(thr_v, jbv * _spl_i(L))
      kk_old = gat(thr_v, jbv * _spl_i(L) + _spl_i(1))
      kkv = jnp.where(is0v, jnp.full((L,), ki, _I), kk_old)
      def tot_i(i, _):
        def tadd(s2, acc):
          return acc + gat(red_v, spl(s2 * 256 + i * L) + iot)
        t = lax.fori_loop(0, NS, tadd, _spl_i(0))
        sca(totf_v, spl(i * L) + iot, t)
        return 0
      lax.fori_loop(0, 256 // L, tot_i, 0)
      def chunk(ic, carry):
        crun, found, gsel, kksel = carry
        i = _spl_i(255 // L) - ic
        tvec = gat(totf_v, i * _spl_i(L) + iot)
        rev = lax.rev(tvec, (0,))
        cs = plsc.cumsum(rev)
        tot = jnp.full((L,), jnp.max(cs), _I)
        candv = crun + cs
        hitm = candv >= kkv
        anyhit = jnp.full((L,), jnp.max(hitm.astype(_I)), _I) > _spl_i(0)
        ffs = plsc.all_reduce_ffs(hitm)
        gcand = i * _spl_i(L) + _spl_i(L - 1) - ffs
        cnt_at = jnp.full((L,), _lane_extract_i(rev, ffs), _I)
        cum_at = jnp.full((L,), _lane_extract_i(candv, ffs), _I)
        kkcand = kkv - (cum_at - cnt_at)
        take = jnp.logical_not(found) & anyhit
        gsel = jnp.where(take, gcand, gsel)
        kksel = jnp.where(take, kkcand, kksel)
        found = found | anyhit
        crun = crun + tot
        return crun, found, gsel, kksel
      crun, found, gsel, kksel = lax.fori_loop(
          0, 256 // L, chunk,
          (_spl_i(0), jnp.full((L,), False, jnp.bool_), _spl_i(0),
           _spl_i(0)))
      pnew = jnp.where(is0v, gsel,
                       lax.shift_left(pref_old, _spl_i(8)) | gsel)
      outv = jnp.where(iot == _spl_i(0), pnew,
                       jnp.where(iot == _spl_i(1), kksel, _spl_i(0)))
      sca(scri_v, iot, outv)
      pltpu.sync_copy(scri_v.at[pl.ds(0, L)], thr_s.at[pl.ds(jb * L, L)])
    plsc.subcore_barrier()
    pltpu.sync_copy(thr_s, thr_v)
    return 0

  def zthr(b, _):
    sca(thr_v, spl(b * L) + iot, _spl_i(0))
    return 0
  lax.fori_loop(0, NB, zthr, 0)
  lax.fori_loop(0, 4, round_r, 0)

  def fin_b(b, _):
    tbits = gat(thr_v, spl(b * L))
    tval = plsc.bitcast(tbits, _F)
    bbase = spl(b * VPS)
    def fin_vv(v, acc):
      mvec = gat(msk_v, bbase + v * L + iot)
      return acc + jnp.where(mvec > tval, mvec, _spl_f(0.0))
    acc = lax.fori_loop(0, NV, fin_vv, _spl_f(0.0))
    sca(scr_v, spl(b * L) + iot, acc)
    return 0
  lax.fori_loop(0, NB, fin_b, 0)
  pltpu.sync_copy(scr_v.at[pl.ds(0, NB * L)],
                  fin_s.at[pl.ds(sid * NB * L, NB * L)])
  plsc.subcore_barrier()

  @pl.when(sid == 0)
  def _():
    pltpu.sync_copy(fin_s, fin_v)
    pltpu.sync_copy(sta_s, sta_v)
    def sacc(s2, carry):
      lsum, cesum = carry
      return (lsum + gat(sta_v, spl(s2 * 6 * L + NB * L) + iot),
              cesum + gat(sta_v, spl(s2 * 6 * L + (NB + 1) * L) + iot))
    lsum, cesum = lax.fori_loop(0, NS, sacc, (_spl_f(0.0), _spl_f(0.0)))
    loss_l = jnp.sum(lsum)
    loss_c = jnp.sum(cesum)
    nsum = jnp.float32(0.0)
    for jb in range(NB):
      def npacc(s2, acc):
        return acc + gat(sta_v, spl(s2 * 6 * L + jb * L) + iot)
      npv = lax.fori_loop(0, NS, npacc, _spl_f(0.0))
      nsum = nsum + jnp.sum(npv)
      def facc(s2, acc):
        return acc + gat(fin_v, spl(s2 * NB * L + jb * L) + iot)
      sv = lax.fori_loop(0, NS, facc, _spl_f(0.0))
      tbits = gat(thr_v, _spl_i(jb * L))
      kkrem = gat(thr_v, _spl_i(jb * L + 1))
      tval = plsc.bitcast(tbits, _F)
      loss_c = loss_c + jnp.sum(sv) + jnp.max(kkrem).astype(_F) \
          * jnp.max(tval)
    ovec = jnp.where(iot == _spl_i(0), jnp.full((L,), loss_l, _F),
                     jnp.where(iot == _spl_i(1), jnp.full((L,), loss_c, _F),
                               jnp.where(iot == _spl_i(2),
                                         jnp.full((L,), nsum, _F),
                                         _spl_f(0.0))))
    sca(scr_v, iot, ovec)
    pltpu.sync_copy(scr_v.at[pl.ds(0, L)], out_hbm.at[pl.ds(cid * L, L)])


@jax.jit
def _run(aux, conf_p):
  mesh = plsc.VectorSubcoreMesh(core_axis_name="c", subcore_axis_name="s")
  f = pl.kernel(
      _sc_body,
      out_type=jax.ShapeDtypeStruct((NC * L,), _F),
      mesh=mesh,
      compiler_params=pltpu.CompilerParams(needs_layout_passes=False),
      scratch_types=[
          pltpu.VMEM((DMAR, C), _F),
          pltpu.VMEM((VPS * 4,), _F),
          pltpu.VMEM((VPS * 4,), _F),
          pltpu.VMEM((NB * O * 5,), _F),
          pltpu.VMEM((NB * 8 * L,), _F),
          pltpu.VMEM((NB * VPS,), _F),
          pltpu.VMEM((NB * VPS,), _F),
          pltpu.VMEM((NS * O * L,), _F),
          pltpu.VMEM((NS * O * L,), _F),
          pltpu.VMEM((NB * VPS,), _F),
          pltpu.VMEM((NB * 256,), _I),
          pltpu.VMEM((NS * 256,), _I),
          pltpu.VMEM((256,), _I),
          pltpu.VMEM((NS * 6 * L,), _F),
          pltpu.VMEM((NS * NB * L,), _F),
          pltpu.VMEM((NB * L,), _F),
          pltpu.VMEM((NB * L,), _I),
          pltpu.VMEM((8 * L,), _F),
          pltpu.VMEM((2 * L,), _I),
          pltpu.VMEM_SHARED((NB * NS * O * L,), _F),
          pltpu.VMEM_SHARED((NB * NS * O * L,), _F),
          pltpu.VMEM_SHARED((NB * L,), _F),
          pltpu.VMEM_SHARED((NS * 6 * L,), _F),
          pltpu.VMEM_SHARED((NB * NS * 256,), _I),
          pltpu.VMEM_SHARED((NB * L,), _I),
          pltpu.VMEM_SHARED((NS * NB * L,), _F),
      ],
  )
  return f(aux, conf_p)


def kernel(loc_data, conf_data, priors, targets):
  aux = jnp.concatenate([
      jnp.pad(loc_data, ((0, 0), (0, 8960 - P), (0, 0))).reshape(-1),
      jnp.pad(priors, ((0, 8960 - P), (0, 0))).reshape(-1),
      targets.reshape(-1)])
  out = _run(aux, conf_data)
  loss_l = out[0] + out[L]
  loss_c = out[1] + out[L + 1]
  n = out[2] + out[L + 2]
  return loss_l / n, loss_c / n
